# baseline (speedup 1.0000x reference)
import jax
import jax.numpy as jnp
from jax import lax
from jax.experimental import pallas as pl
from jax.experimental.pallas import tpu as pltpu
from jax.experimental.pallas import tpu_sc as plsc

_NC = 2
_NW = 32


def _ceil32(n):
    return (n + 31) // 32 * 32


def _pick_T(K, D, weighted, Np):
    for T in (32, 16):
        KT = K * T
        need = (2 + (1 if weighted else 0)) * KT * D * 4 + 2 * T * D * 4
        if T == 32 and Np // 32 < 12:
            continue
        if need <= 420_000 and Np % T == 0:
            return T
    return 16


def _sc_gather_sum(table, pidx, wexp, bias, *, K, D, Np, B, rowstride, relu):
    weighted0 = wexp is not None
    T = _pick_T(K, D, weighted0, Np)
    KT = K * T
    nchunks = Np // T
    BG = 2 if nchunks >= 32 else (4 if nchunks >= 16 else 8)
    bsz = B // BG
    P = bsz // 2
    njobs = nchunks * BG
    per = -(-njobs // _NW)
    parts = []
    off = 0
    while off < KT:
        r = min(128, KT - off)
        parts.append((off, r))
        off += r
    weighted = wexp is not None
    has_bias = bias is not None
    is_bf16 = table.dtype == jnp.bfloat16
    tdt = jnp.bfloat16 if is_bf16 else jnp.float32

    scratch = [pltpu.VMEM((KT,), jnp.int32)]
    for _ in range(2):
        for (_o, r) in parts:
            scratch.append(pltpu.VMEM((r,), jnp.int32))
    scratch += [pltpu.VMEM((KT, D), tdt),
                pltpu.VMEM((KT, D), tdt)]
    if weighted:
        scratch.append(pltpu.VMEM((KT, D), jnp.float32))
    if has_bias:
        scratch.append(pltpu.VMEM((D,), tdt))
    scratch += [pltpu.VMEM((T, D), jnp.float32),
                pltpu.VMEM((T, D), jnp.float32)]
    scratch += [pltpu.SemaphoreType.DMA] * 4

    mesh = plsc.VectorSubcoreMesh(core_axis_name="c", subcore_axis_name="s")

    def body(*refs):
        it = iter(refs)
        table_r = next(it)
        pidx_r = next(it)
        wexp_r = next(it) if weighted else None
        bias_r = next(it) if has_bias else None
        out_r = next(it)
        pidxv = next(it)
        idxb = [[next(it) for _ in parts], [next(it) for _ in parts]]
        gbuf = [next(it), next(it)]
        wbuf = next(it) if weighted else None
        biasv = next(it) if has_bias else None
        obuf = [next(it), next(it)]
        sem_g = [next(it), next(it)]
        sem_s = [next(it), next(it)]

        wid = lax.axis_index("s") * _NC + lax.axis_index("c")
        if has_bias:
            pltpu.sync_copy(bias_r, biasv)

        def build_idx(b, u):
            boff = b * rowstride
            for v in range(KT // 16):
                pi = v // 8
                o_local = 16 * v - parts[pi][0]
                idxb[u][pi][pl.ds(o_local, 16)] = \
                    pidxv[pl.ds(16 * v, 16)] + boff

        def fire_gather(u):
            for (o, r), iref in zip(parts, idxb[u]):
                pltpu.async_copy(table_r.at[iref],
                                 gbuf[u].at[pl.ds(o, r)], sem_g[u])

        def drain_gather(u):
            for (o, r), iref in zip(parts, idxb[u]):
                pltpu.make_async_copy(table_r.at[iref],
                                      gbuf[u].at[pl.ds(o, r)],
                                      sem_g[u]).wait()

        def fire_scatter(u, b, n0):
            pltpu.async_copy(obuf[u], out_r.at[pl.ds(b * Np + n0, T)],
                             sem_s[u])

        def drain_scatter(u):
            pltpu.make_async_copy(obuf[u], out_r.at[pl.ds(0, T)],
                                  sem_s[u]).wait()

        def _tree(vals):
            while len(vals) > 1:
                nxt = [vals[i] + vals[i + 1]
                       for i in range(0, len(vals) - 1, 2)]
                if len(vals) % 2:
                    nxt.append(vals[-1])
                vals = nxt
            return vals[0]

        def _unpack(g32):
            u = plsc.bitcast(g32, jnp.uint32)
            lo = plsc.bitcast(u << 16, jnp.float32)
            hi = plsc.bitcast(u & jnp.uint32(0xFFFF0000), jnp.float32)
            return lo, hi

        def compute(u):
            gb, ob = gbuf[u], obuf[u]

            @plsc.parallel_loop(0, T, 1, unroll=2)
            def _tbody(t):
                row = t * K
                if is_bf16:
                    for dd in range(D // 32):
                        o = dd * 32
                        los, his = [], []
                        for k in range(K):
                            lo, hi = _unpack(gb[row + k, pl.ds(o, 32)])
                            los.append(lo)
                            his.append(hi)
                        acc_lo = _tree(los)
                        acc_hi = _tree(his)
                        if has_bias:
                            blo, bhi = _unpack(biasv[pl.ds(o, 32)])
                            acc_lo = acc_lo + blo
                            acc_hi = acc_hi + bhi
                        if relu:
                            acc_lo = jnp.maximum(acc_lo, 0.0)
                            acc_hi = jnp.maximum(acc_hi, 0.0)
                        ob[t, pl.ds(o, 16)] = acc_lo
                        ob[t, pl.ds(o + 16, 16)] = acc_hi
                    return
                for dd in range(D // 16):
                    o = dd * 16
                    vals = []
                    for k in range(K):
                        g = gb[row + k, pl.ds(o, 16)]
                        if weighted:
                            g = g * wbuf[row + k, pl.ds(o, 16)]
                        vals.append(g)
                    acc = _tree(vals)
                    if has_bias:
                        acc = acc + biasv[pl.ds(o, 16)]
                    if relu:
                        acc = jnp.maximum(acc, 0.0)
                    ob[t, pl.ds(o, 16)] = acc

        def job(jw, _):
            j = jw * _NW + wid

            @pl.when(j < njobs)
            def _():
                ci = j // BG
                bg = j % BG
                n0 = ci * T
                base = bg * bsz
                pltpu.sync_copy(pidx_r.at[pl.ds(n0 * K, KT)], pidxv)
                if weighted:
                    pltpu.sync_copy(wexp_r.at[pl.ds(n0 * K, KT)], wbuf)
                build_idx(base, 0)
                fire_gather(0)

                def pair(bp, _):
                    b0 = base + 2 * bp
                    build_idx(b0 + 1, 1)
                    fire_gather(1)
                    drain_gather(0)

                    @pl.when(bp > 0)
                    def _():
                        drain_scatter(0)

                    compute(0)
                    fire_scatter(0, b0, n0)

                    @pl.when(bp < P - 1)
                    def _():
                        build_idx(b0 + 2, 0)
                        fire_gather(0)

                    drain_gather(1)

                    @pl.when(bp > 0)
                    def _():
                        drain_scatter(1)

                    compute(1)
                    fire_scatter(1, b0 + 1, n0)
                    return 0

                lax.fori_loop(0, P, pair, 0)
                drain_scatter(0)
                drain_scatter(1)

            return 0

        lax.fori_loop(0, per, job, 0)

    args = [table, pidx]
    if weighted:
        args.append(wexp)
    if has_bias:
        args.append(bias.astype(tdt))
    fn = pl.kernel(body,
                   out_type=jax.ShapeDtypeStruct((B * Np, D), jnp.float32),
                   mesh=mesh,
                   scratch_types=scratch,
                   compiler_params=pltpu.CompilerParams(
                       use_tc_tiling_on_sc=False,
                       needs_layout_passes=False))
    return fn(*args)


def _mm_body(x_ref, w_ref, o_ref):
    o_ref[...] = jnp.dot(x_ref[...], w_ref[...],
                         preferred_element_type=jnp.float32
                         ).astype(o_ref.dtype)


def _tc_matmul(x, w, out_dtype=jnp.float32):
    M, C = x.shape
    KO = w.shape[1]
    BM = 256
    return pl.pallas_call(
        _mm_body,
        grid=(M // BM,),
        in_specs=[pl.BlockSpec((BM, C), lambda i: (i, 0)),
                  pl.BlockSpec((C, KO), lambda i: (0, 0))],
        out_specs=pl.BlockSpec((BM, KO), lambda i: (i, 0)),
        out_shape=jax.ShapeDtypeStruct((M, KO), out_dtype),
    )(x, w)


def _bilin_body(uv_ref, x_ref, up_ref, o_ref):
    uvb = uv_ref[0]
    im = x_ref[0]
    up = up_ref[...]
    uvc = jnp.clip((uvb - 0.5) * 2.0, -1.0, 1.0)
    gx = (uvc[:, 0:1] + 1.0) * 1.5
    gy = (uvc[:, 1:2] + 1.0) * 1.5
    x0 = jnp.floor(gx)
    y0 = jnp.floor(gy)
    wa = (x0 + 1.0 - gx) * (y0 + 1.0 - gy)
    wb = (x0 + 1.0 - gx) * (gy - y0)
    wc = (gx - x0) * (y0 + 1.0 - gy)
    wd = (gx - x0) * (gy - y0)
    x0i = x0.astype(jnp.int32)
    y0i = y0.astype(jnp.int32)
    cell = lax.broadcasted_iota(jnp.int32, (21, 16), 1)

    def oh(xi, yi, w):
        c = yi * 4 + xi
        valid = (xi >= 0) & (xi <= 3) & (yi >= 0) & (yi <= 3)
        return jnp.where((cell == c) & valid, w, 0.0)

    wgt = (oh(x0i, y0i, wa) + oh(x0i, y0i + 1, wb) +
           oh(x0i + 1, y0i, wc) + oh(x0i + 1, y0i + 1, wd))
    t1 = lax.dot_general(wgt, im, (((1,), (1,)), ((), ())),
                         preferred_element_type=jnp.float32)
    o_ref[0] = jnp.dot(up, t1, preferred_element_type=jnp.float32)


def _tc_bilinear_upsample(uv, x16, upsample):
    B = uv.shape[0]
    return pl.pallas_call(
        _bilin_body,
        grid=(B,),
        in_specs=[pl.BlockSpec((1, 21, 2), lambda b: (b, 0, 0)),
                  pl.BlockSpec((1, 256, 16), lambda b: (b, 0, 0)),
                  pl.BlockSpec((98, 21), lambda b: (0, 0))],
        out_specs=pl.BlockSpec((1, 98, 256), lambda b: (b, 0, 0)),
        out_shape=jax.ShapeDtypeStruct((B, 98, 256), jnp.float32),
    )(uv, x16, upsample)


def _wcat(pw, dw, S, C, O, Opad=None):
    pwdw = (pw * dw[None, :]).reshape(O, S, C)
    w = jnp.transpose(pwdw, (2, 1, 0))
    if Opad is not None and Opad != O:
        w = jnp.pad(w, ((0, 0), (0, 0), (0, Opad - O)))
        O = Opad
    return w.reshape(C, S * O)


def _bf16_perm(C):
    perm = []
    for g in range(C // 32):
        base = 32 * g
        perm += [base + 2 * w for w in range(16)]
        perm += [base + 2 * w + 1 for w in range(16)]
    return perm


def _half(uv, x, upsample, idx0, idx1, idx2, idx3, col0, col1, col2, col3,
          val0, val1, val2, val3, rmap0, rmap1, rmap2, rmap3,
          dw0, pw0, b0, dw1, pw1, b1, dw2, pw2, b2, dw3, pw3, b3,
          dwh, pwh, bh):
    B = uv.shape[0]
    f32 = jnp.float32

    z = _tc_bilinear_upsample(uv, x.reshape(B, 256, 16), upsample)
    z = z.reshape(B * 98, 256)
    vtab = 98
    perm_in = list(range(256))

    levels = [
        (col3, val3, rmap3, idx3, dw0, pw0, b0),
        (col2, val2, rmap2, idx2, dw1, pw1, b1),
        (col1, val1, rmap1, idx1, dw2, pw2, b2),
        (col0, val0, rmap0, idx0, dw3, pw3, b3),
    ]
    for col, val, rmap, idx, dw, pw, b in levels:
        N, S = idx.shape
        Np = _ceil32(N)
        O = pw.shape[0]
        C = dw.shape[0] // S
        pcol = jnp.pad(jnp.take(col, rmap), ((0, Np - N), (0, 0)))
        pval = jnp.pad(jnp.take(val, rmap), ((0, Np - N), (0, 0)))
        pidx = pcol.reshape(-1).astype(jnp.int32)
        wexp = jnp.broadcast_to(pval.reshape(-1)[:, None].astype(f32),
                                (Np * 3, C))
        X = _sc_gather_sum(z, pidx, wexp, None, K=3, D=C, Np=Np, B=B,
                           rowstride=vtab, relu=False)
        Wc = _wcat(pw, dw, S, C, O)[jnp.asarray(perm_in, dtype=jnp.int32)]
        Y = _tc_matmul(X, Wc, out_dtype=jnp.bfloat16)
        Yr = Y.reshape(B * Np * S, O)
        bidx = idx.astype(jnp.int32) * S + jnp.arange(S, dtype=jnp.int32)[None]
        bidx = jnp.pad(bidx, ((0, Np - N), (0, 0))).reshape(-1)
        z = _sc_gather_sum(Yr, bidx, None, b.astype(f32), K=S, D=O, Np=Np,
                           B=B, rowstride=Np * S, relu=True)
        vtab = Np
        perm_in = _bf16_perm(O)

    N, S = idx0.shape
    Opad = 32
    Wh = _wcat(pwh, dwh, S, 32, 3, Opad=Opad)[jnp.asarray(perm_in,
                                                          dtype=jnp.int32)]
    Yh = _tc_matmul(z, Wh, out_dtype=jnp.bfloat16)
    Yhr = Yh.reshape(B * vtab * S, Opad)
    bidxh = (idx0.astype(jnp.int32) * S +
             jnp.arange(S, dtype=jnp.int32)[None]).reshape(-1)
    out = _sc_gather_sum(Yhr, bidxh, None,
                         jnp.pad(bh.astype(f32), (0, Opad - 3)),
                         K=S, D=Opad, Np=N, B=B, rowstride=vtab * S,
                         relu=False)
    return out.reshape(B, N, Opad)[:, :, jnp.asarray((0, 16, 1))]


def kernel(uv, x, upsample, idx0, idx1, idx2, idx3, col0, col1, col2, col3,
           val0, val1, val2, val3, rmap0, rmap1, rmap2, rmap3,
           dw0, pw0, b0, dw1, pw1, b1, dw2, pw2, b2, dw3, pw3, b3,
           dwh, pwh, bh):
    B = uv.shape[0]
    H = B // 2
    rest = (idx0, idx1, idx2, idx3, col0, col1, col2, col3,
            val0, val1, val2, val3, rmap0, rmap1, rmap2, rmap3,
            dw0, pw0, b0, dw1, pw1, b1, dw2, pw2, b2, dw3, pw3, b3,
            dwh, pwh, bh)
    o1 = _half(uv[:H], x[:H], upsample, *rest)
    o2 = _half(uv[H:], x[H:], upsample, *rest)
    return jnp.concatenate([o1, o2], axis=0)


# speedup vs baseline: 1.2185x; 1.2185x over previous
import jax
import jax.numpy as jnp
from jax import lax
from jax.experimental import pallas as pl
from jax.experimental.pallas import tpu as pltpu
from jax.experimental.pallas import tpu_sc as plsc

_NC = 2
_NW = 32


def _ceil32(n):
    return (n + 31) // 32 * 32


def _pick_T(K, D, weighted, Np):
    for T in (32, 16):
        KT = K * T
        need = (2 + (1 if weighted else 0)) * KT * D * 4 + 2 * T * D * 4
        if T == 32 and Np // 32 < 12:
            continue
        if need <= 420_000 and Np % T == 0:
            return T
    return 16


def _sc_gather_sum(table, pidx, wexp, bias, *, K, D, Np, B, rowstride, relu):
    weighted0 = wexp is not None
    T = _pick_T(K, D, weighted0, Np)
    KT = K * T
    nchunks = Np // T
    BG = 2 if nchunks >= 32 else (4 if nchunks >= 16 else 8)
    bsz = B // BG
    P = bsz // 2
    njobs = nchunks * BG
    per = -(-njobs // _NW)
    parts = []
    off = 0
    while off < KT:
        r = min(128, KT - off)
        parts.append((off, r))
        off += r
    weighted = wexp is not None
    has_bias = bias is not None

    scratch = [pltpu.VMEM((KT,), jnp.int32)]
    for _ in range(2):
        for (_o, r) in parts:
            scratch.append(pltpu.VMEM((r,), jnp.int32))
    scratch += [pltpu.VMEM((KT, D), jnp.float32),
                pltpu.VMEM((KT, D), jnp.float32)]
    if weighted:
        scratch.append(pltpu.VMEM((KT, D), jnp.float32))
    if has_bias:
        scratch.append(pltpu.VMEM((D,), jnp.float32))
    scratch += [pltpu.VMEM((T, D), jnp.float32),
                pltpu.VMEM((T, D), jnp.float32)]
    scratch += [pltpu.SemaphoreType.DMA] * 4

    mesh = plsc.VectorSubcoreMesh(core_axis_name="c", subcore_axis_name="s")

    def body(*refs):
        it = iter(refs)
        table_r = next(it)
        pidx_r = next(it)
        wexp_r = next(it) if weighted else None
        bias_r = next(it) if has_bias else None
        out_r = next(it)
        pidxv = next(it)
        idxb = [[next(it) for _ in parts], [next(it) for _ in parts]]
        gbuf = [next(it), next(it)]
        wbuf = next(it) if weighted else None
        biasv = next(it) if has_bias else None
        obuf = [next(it), next(it)]
        sem_g = [next(it), next(it)]
        sem_s = [next(it), next(it)]

        wid = lax.axis_index("s") * _NC + lax.axis_index("c")
        if has_bias:
            pltpu.sync_copy(bias_r, biasv)

        def build_idx(b, u):
            boff = b * rowstride
            for v in range(KT // 16):
                pi = v // 8
                o_local = 16 * v - parts[pi][0]
                idxb[u][pi][pl.ds(o_local, 16)] = \
                    pidxv[pl.ds(16 * v, 16)] + boff

        def fire_gather(u):
            for (o, r), iref in zip(parts, idxb[u]):
                pltpu.async_copy(table_r.at[iref],
                                 gbuf[u].at[pl.ds(o, r)], sem_g[u])

        def drain_gather(u):
            for (o, r), iref in zip(parts, idxb[u]):
                pltpu.make_async_copy(table_r.at[iref],
                                      gbuf[u].at[pl.ds(o, r)],
                                      sem_g[u]).wait()

        def fire_scatter(u, b, n0):
            pltpu.async_copy(obuf[u], out_r.at[pl.ds(b * Np + n0, T)],
                             sem_s[u])

        def drain_scatter(u):
            pltpu.make_async_copy(obuf[u], out_r.at[pl.ds(0, T)],
                                  sem_s[u]).wait()

        def compute(u):
            gb, ob = gbuf[u], obuf[u]

            @plsc.parallel_loop(0, T, 1, unroll=4)
            def _tbody(t):
                row = t * K
                for dd in range(D // 16):
                    o = dd * 16
                    vals = []
                    for k in range(K):
                        g = gb[row + k, pl.ds(o, 16)]
                        if weighted:
                            g = g * wbuf[row + k, pl.ds(o, 16)]
                        vals.append(g)
                    while len(vals) > 1:
                        nxt = [vals[i] + vals[i + 1]
                               for i in range(0, len(vals) - 1, 2)]
                        if len(vals) % 2:
                            nxt.append(vals[-1])
                        vals = nxt
                    acc = vals[0]
                    if has_bias:
                        acc = acc + biasv[pl.ds(o, 16)]
                    if relu:
                        acc = jnp.maximum(acc, 0.0)
                    ob[t, pl.ds(o, 16)] = acc

        def job(jw, _):
            j = jw * _NW + wid

            @pl.when(j < njobs)
            def _():
                ci = j // BG
                bg = j % BG
                n0 = ci * T
                base = bg * bsz
                pltpu.sync_copy(pidx_r.at[pl.ds(n0 * K, KT)], pidxv)
                if weighted:
                    pltpu.sync_copy(wexp_r.at[pl.ds(n0 * K, KT)], wbuf)
                build_idx(base, 0)
                fire_gather(0)

                def pair(bp, _):
                    b0 = base + 2 * bp
                    build_idx(b0 + 1, 1)
                    fire_gather(1)
                    drain_gather(0)

                    @pl.when(bp > 0)
                    def _():
                        drain_scatter(0)

                    compute(0)
                    fire_scatter(0, b0, n0)

                    @pl.when(bp < P - 1)
                    def _():
                        build_idx(b0 + 2, 0)
                        fire_gather(0)

                    drain_gather(1)

                    @pl.when(bp > 0)
                    def _():
                        drain_scatter(1)

                    compute(1)
                    fire_scatter(1, b0 + 1, n0)
                    return 0

                lax.fori_loop(0, P, pair, 0)
                drain_scatter(0)
                drain_scatter(1)

            return 0

        lax.fori_loop(0, per, job, 0)

    args = [table, pidx]
    if weighted:
        args.append(wexp)
    if has_bias:
        args.append(bias)
    fn = pl.kernel(body,
                   out_type=jax.ShapeDtypeStruct((B * Np, D), jnp.float32),
                   mesh=mesh,
                   scratch_types=scratch,
                   compiler_params=pltpu.CompilerParams(
                       use_tc_tiling_on_sc=False))
    return fn(*args)


def _mm_body(x_ref, w_ref, o_ref):
    o_ref[...] = jnp.dot(x_ref[...], w_ref[...],
                         preferred_element_type=jnp.float32)


def _tc_matmul(x, w):
    M, C = x.shape
    KO = w.shape[1]
    BM = 256
    return pl.pallas_call(
        _mm_body,
        grid=(M // BM,),
        in_specs=[pl.BlockSpec((BM, C), lambda i: (i, 0)),
                  pl.BlockSpec((C, KO), lambda i: (0, 0))],
        out_specs=pl.BlockSpec((BM, KO), lambda i: (i, 0)),
        out_shape=jax.ShapeDtypeStruct((M, KO), jnp.float32),
    )(x, w)


def _bilin_body(uv_ref, x_ref, up_ref, o_ref):
    uvb = uv_ref[0]
    im = x_ref[0]
    up = up_ref[...]
    uvc = jnp.clip((uvb - 0.5) * 2.0, -1.0, 1.0)
    gx = (uvc[:, 0:1] + 1.0) * 1.5
    gy = (uvc[:, 1:2] + 1.0) * 1.5
    x0 = jnp.floor(gx)
    y0 = jnp.floor(gy)
    wa = (x0 + 1.0 - gx) * (y0 + 1.0 - gy)
    wb = (x0 + 1.0 - gx) * (gy - y0)
    wc = (gx - x0) * (y0 + 1.0 - gy)
    wd = (gx - x0) * (gy - y0)
    x0i = x0.astype(jnp.int32)
    y0i = y0.astype(jnp.int32)
    cell = lax.broadcasted_iota(jnp.int32, (21, 16), 1)

    def oh(xi, yi, w):
        c = yi * 4 + xi
        valid = (xi >= 0) & (xi <= 3) & (yi >= 0) & (yi <= 3)
        return jnp.where((cell == c) & valid, w, 0.0)

    wgt = (oh(x0i, y0i, wa) + oh(x0i, y0i + 1, wb) +
           oh(x0i + 1, y0i, wc) + oh(x0i + 1, y0i + 1, wd))
    t1 = lax.dot_general(wgt, im, (((1,), (1,)), ((), ())),
                         preferred_element_type=jnp.float32)
    o_ref[0] = jnp.dot(up, t1, preferred_element_type=jnp.float32)


def _tc_bilinear_upsample(uv, x16, upsample):
    B = uv.shape[0]
    return pl.pallas_call(
        _bilin_body,
        grid=(B,),
        in_specs=[pl.BlockSpec((1, 21, 2), lambda b: (b, 0, 0)),
                  pl.BlockSpec((1, 256, 16), lambda b: (b, 0, 0)),
                  pl.BlockSpec((98, 21), lambda b: (0, 0))],
        out_specs=pl.BlockSpec((1, 98, 256), lambda b: (b, 0, 0)),
        out_shape=jax.ShapeDtypeStruct((B, 98, 256), jnp.float32),
    )(uv, x16, upsample)


def _wcat(pw, dw, S, C, O, Opad=None):
    pwdw = (pw * dw[None, :]).reshape(O, S, C)
    w = jnp.transpose(pwdw, (2, 1, 0))
    if Opad is not None and Opad != O:
        w = jnp.pad(w, ((0, 0), (0, 0), (0, Opad - O)))
        O = Opad
    return w.reshape(C, S * O)


def _half(uv, x, upsample, idx0, idx1, idx2, idx3, col0, col1, col2, col3,
          val0, val1, val2, val3, rmap0, rmap1, rmap2, rmap3,
          dw0, pw0, b0, dw1, pw1, b1, dw2, pw2, b2, dw3, pw3, b3,
          dwh, pwh, bh):
    B = uv.shape[0]
    f32 = jnp.float32

    z = _tc_bilinear_upsample(uv, x.reshape(B, 256, 16), upsample)
    z = z.reshape(B * 98, 256)
    vtab = 98

    levels = [
        (col3, val3, rmap3, idx3, dw0, pw0, b0),
        (col2, val2, rmap2, idx2, dw1, pw1, b1),
        (col1, val1, rmap1, idx1, dw2, pw2, b2),
        (col0, val0, rmap0, idx0, dw3, pw3, b3),
    ]
    for col, val, rmap, idx, dw, pw, b in levels:
        N, S = idx.shape
        Np = _ceil32(N)
        O = pw.shape[0]
        C = dw.shape[0] // S
        pcol = jnp.pad(jnp.take(col, rmap), ((0, Np - N), (0, 0)))
        pval = jnp.pad(jnp.take(val, rmap), ((0, Np - N), (0, 0)))
        pidx = pcol.reshape(-1).astype(jnp.int32)
        wexp = jnp.broadcast_to(pval.reshape(-1)[:, None].astype(f32),
                                (Np * 3, C))
        X = _sc_gather_sum(z, pidx, wexp, None, K=3, D=C, Np=Np, B=B,
                           rowstride=vtab, relu=False)
        Y = _tc_matmul(X, _wcat(pw, dw, S, C, O))
        Yr = Y.reshape(B * Np * S, O)
        bidx = idx.astype(jnp.int32) * S + jnp.arange(S, dtype=jnp.int32)[None]
        bidx = jnp.pad(bidx, ((0, Np - N), (0, 0))).reshape(-1)
        z = _sc_gather_sum(Yr, bidx, None, b.astype(f32), K=S, D=O, Np=Np,
                           B=B, rowstride=Np * S, relu=True)
        vtab = Np

    N, S = idx0.shape
    Opad = 16
    Yh = _tc_matmul(z, _wcat(pwh, dwh, S, 32, 3, Opad=Opad))
    Yhr = Yh.reshape(B * vtab * S, Opad)
    bidxh = (idx0.astype(jnp.int32) * S +
             jnp.arange(S, dtype=jnp.int32)[None]).reshape(-1)
    out = _sc_gather_sum(Yhr, bidxh, None,
                         jnp.pad(bh.astype(f32), (0, Opad - 3)),
                         K=S, D=Opad, Np=N, B=B, rowstride=vtab * S,
                         relu=False)
    return out.reshape(B, N, Opad)[:, :, :3]


def kernel(uv, x, upsample, idx0, idx1, idx2, idx3, col0, col1, col2, col3,
           val0, val1, val2, val3, rmap0, rmap1, rmap2, rmap3,
           dw0, pw0, b0, dw1, pw1, b1, dw2, pw2, b2, dw3, pw3, b3,
           dwh, pwh, bh):
    B = uv.shape[0]
    H = B // 2
    rest = (idx0, idx1, idx2, idx3, col0, col1, col2, col3,
            val0, val1, val2, val3, rmap0, rmap1, rmap2, rmap3,
            dw0, pw0, b0, dw1, pw1, b1, dw2, pw2, b2, dw3, pw3, b3,
            dwh, pwh, bh)
    o1 = _half(uv[:H], x[:H], upsample, *rest)
    o2 = _half(uv[H:], x[H:], upsample, *rest)
    return jnp.concatenate([o1, o2], axis=0)


# speedup vs baseline: 1.3022x; 1.0687x over previous
import jax
import jax.numpy as jnp
from jax import lax
from jax.experimental import pallas as pl
from jax.experimental.pallas import tpu as pltpu
from jax.experimental.pallas import tpu_sc as plsc

_NC = 2
_NW = 32


def _ceil32(n):
    return (n + 15) // 16 * 16


def _pick_T(K, D, weighted, Np):
    return 16


def _sc_gather_sum(table, pidx, wexp, bias, *, K, D, Np, B, rowstride, relu):
    weighted0 = wexp is not None
    T = _pick_T(K, D, weighted0, Np)
    KT = K * T
    nchunks = Np // T
    BG = 2 if nchunks >= 32 else (4 if nchunks >= 16 else 8)
    bsz = B // BG
    P = bsz // 2
    njobs = nchunks * BG
    per = -(-njobs // _NW)
    parts = []
    off = 0
    while off < KT:
        r = min(128, KT - off)
        parts.append((off, r))
        off += r
    weighted = wexp is not None
    has_bias = bias is not None

    scratch = [pltpu.VMEM((KT,), jnp.int32)]
    for _ in range(2):
        for (_o, r) in parts:
            scratch.append(pltpu.VMEM((r,), jnp.int32))
    scratch += [pltpu.VMEM((KT, D), jnp.float32),
                pltpu.VMEM((KT, D), jnp.float32)]
    if weighted:
        scratch.append(pltpu.VMEM((KT, D), jnp.float32))
    if has_bias:
        scratch.append(pltpu.VMEM((D,), jnp.float32))
    scratch += [pltpu.VMEM((T, D), jnp.float32),
                pltpu.VMEM((T, D), jnp.float32)]
    scratch += [pltpu.SemaphoreType.DMA] * 4

    mesh = plsc.VectorSubcoreMesh(core_axis_name="c", subcore_axis_name="s")

    def body(*refs):
        it = iter(refs)
        table_r = next(it)
        pidx_r = next(it)
        wexp_r = next(it) if weighted else None
        bias_r = next(it) if has_bias else None
        out_r = next(it)
        pidxv = next(it)
        idxb = [[next(it) for _ in parts], [next(it) for _ in parts]]
        gbuf = [next(it), next(it)]
        wbuf = next(it) if weighted else None
        biasv = next(it) if has_bias else None
        obuf = [next(it), next(it)]
        sem_g = [next(it), next(it)]
        sem_s = [next(it), next(it)]

        wid = lax.axis_index("s") * _NC + lax.axis_index("c")
        if has_bias:
            pltpu.sync_copy(bias_r, biasv)

        def build_idx(b, u):
            boff = b * rowstride
            for v in range(KT // 16):
                pi = v // 8
                o_local = 16 * v - parts[pi][0]
                idxb[u][pi][pl.ds(o_local, 16)] = \
                    pidxv[pl.ds(16 * v, 16)] + boff

        def fire_gather(u):
            for (o, r), iref in zip(parts, idxb[u]):
                pltpu.async_copy(table_r.at[iref],
                                 gbuf[u].at[pl.ds(o, r)], sem_g[u])

        def drain_gather(u):
            for (o, r), iref in zip(parts, idxb[u]):
                pltpu.make_async_copy(table_r.at[iref],
                                      gbuf[u].at[pl.ds(o, r)],
                                      sem_g[u]).wait()

        def fire_scatter(u, b, n0):
            pltpu.async_copy(obuf[u], out_r.at[pl.ds(b * Np + n0, T)],
                             sem_s[u])

        def drain_scatter(u):
            pltpu.make_async_copy(obuf[u], out_r.at[pl.ds(0, T)],
                                  sem_s[u]).wait()

        def compute(u):
            gb, ob = gbuf[u], obuf[u]

            @plsc.parallel_loop(0, T, 1, unroll=2)
            def _tbody(t):
                row = t * K
                for dd in range(D // 16):
                    o = dd * 16
                    vals = []
                    for k in range(K):
                        g = gb[row + k, pl.ds(o, 16)]
                        if weighted:
                            g = g * wbuf[row + k, pl.ds(o, 16)]
                        vals.append(g)
                    while len(vals) > 1:
                        nxt = [vals[i] + vals[i + 1]
                               for i in range(0, len(vals) - 1, 2)]
                        if len(vals) % 2:
                            nxt.append(vals[-1])
                        vals = nxt
                    acc = vals[0]
                    if has_bias:
                        acc = acc + biasv[pl.ds(o, 16)]
                    if relu:
                        acc = jnp.maximum(acc, 0.0)
                    ob[t, pl.ds(o, 16)] = acc

        def job(jw, _):
            j = jw * _NW + wid

            @pl.when(j < njobs)
            def _():
                ci = j // BG
                bg = j % BG
                n0 = ci * T
                base = bg * bsz
                pltpu.sync_copy(pidx_r.at[pl.ds(n0 * K, KT)], pidxv)
                if weighted:
                    pltpu.sync_copy(wexp_r.at[pl.ds(n0 * K, KT)], wbuf)
                build_idx(base, 0)
                fire_gather(0)

                def pair(bp, _):
                    b0 = base + 2 * bp
                    build_idx(b0 + 1, 1)
                    fire_gather(1)
                    drain_gather(0)

                    @pl.when(bp > 0)
                    def _():
                        drain_scatter(0)

                    compute(0)
                    fire_scatter(0, b0, n0)

                    @pl.when(bp < P - 1)
                    def _():
                        build_idx(b0 + 2, 0)
                        fire_gather(0)

                    drain_gather(1)

                    @pl.when(bp > 0)
                    def _():
                        drain_scatter(1)

                    compute(1)
                    fire_scatter(1, b0 + 1, n0)
                    return 0

                lax.fori_loop(0, P, pair, 0)
                drain_scatter(0)
                drain_scatter(1)

            return 0

        lax.fori_loop(0, per, job, 0)

    args = [table, pidx]
    if weighted:
        args.append(wexp)
    if has_bias:
        args.append(bias)
    fn = pl.kernel(body,
                   out_type=jax.ShapeDtypeStruct((B * Np, D), jnp.float32),
                   mesh=mesh,
                   scratch_types=scratch,
                   compiler_params=pltpu.CompilerParams(
                       use_tc_tiling_on_sc=False))
    return fn(*args)


def _mm_body(x_ref, w_ref, o_ref):
    o_ref[...] = jnp.dot(x_ref[...], w_ref[...],
                         preferred_element_type=jnp.float32)


def _tc_matmul(x, w):
    M, C = x.shape
    KO = w.shape[1]
    BM = 256
    return pl.pallas_call(
        _mm_body,
        grid=(M // BM,),
        in_specs=[pl.BlockSpec((BM, C), lambda i: (i, 0)),
                  pl.BlockSpec((C, KO), lambda i: (0, 0))],
        out_specs=pl.BlockSpec((BM, KO), lambda i: (i, 0)),
        out_shape=jax.ShapeDtypeStruct((M, KO), jnp.float32),
    )(x, w)


def _bilin_body(uv_ref, x_ref, up_ref, o_ref):
    uvb = uv_ref[0]
    im = x_ref[0]
    up = up_ref[...]
    uvc = jnp.clip((uvb - 0.5) * 2.0, -1.0, 1.0)
    gx = (uvc[:, 0:1] + 1.0) * 1.5
    gy = (uvc[:, 1:2] + 1.0) * 1.5
    x0 = jnp.floor(gx)
    y0 = jnp.floor(gy)
    wa = (x0 + 1.0 - gx) * (y0 + 1.0 - gy)
    wb = (x0 + 1.0 - gx) * (gy - y0)
    wc = (gx - x0) * (y0 + 1.0 - gy)
    wd = (gx - x0) * (gy - y0)
    x0i = x0.astype(jnp.int32)
    y0i = y0.astype(jnp.int32)
    cell = lax.broadcasted_iota(jnp.int32, (21, 16), 1)

    def oh(xi, yi, w):
        c = yi * 4 + xi
        valid = (xi >= 0) & (xi <= 3) & (yi >= 0) & (yi <= 3)
        return jnp.where((cell == c) & valid, w, 0.0)

    wgt = (oh(x0i, y0i, wa) + oh(x0i, y0i + 1, wb) +
           oh(x0i + 1, y0i, wc) + oh(x0i + 1, y0i + 1, wd))
    t1 = lax.dot_general(wgt, im, (((1,), (1,)), ((), ())),
                         preferred_element_type=jnp.float32)
    o_ref[0] = jnp.dot(up, t1, preferred_element_type=jnp.float32)


def _tc_bilinear_upsample(uv, x16, upsample):
    B = uv.shape[0]
    return pl.pallas_call(
        _bilin_body,
        grid=(B,),
        in_specs=[pl.BlockSpec((1, 21, 2), lambda b: (b, 0, 0)),
                  pl.BlockSpec((1, 256, 16), lambda b: (b, 0, 0)),
                  pl.BlockSpec((98, 21), lambda b: (0, 0))],
        out_specs=pl.BlockSpec((1, 98, 256), lambda b: (b, 0, 0)),
        out_shape=jax.ShapeDtypeStruct((B, 98, 256), jnp.float32),
    )(uv, x16, upsample)


def _wcat(pw, dw, S, C, O, Opad=None):
    pwdw = (pw * dw[None, :]).reshape(O, S, C)
    w = jnp.transpose(pwdw, (2, 1, 0))
    if Opad is not None and Opad != O:
        w = jnp.pad(w, ((0, 0), (0, 0), (0, Opad - O)))
        O = Opad
    return w.reshape(C, S * O)


def _half(uv, x, upsample, idx0, idx1, idx2, idx3, col0, col1, col2, col3,
          val0, val1, val2, val3, rmap0, rmap1, rmap2, rmap3,
          dw0, pw0, b0, dw1, pw1, b1, dw2, pw2, b2, dw3, pw3, b3,
          dwh, pwh, bh):
    B = uv.shape[0]
    f32 = jnp.float32

    z = _tc_bilinear_upsample(uv, x.reshape(B, 256, 16), upsample)
    z = z.reshape(B * 98, 256)
    vtab = 98

    levels = [
        (col3, val3, rmap3, idx3, dw0, pw0, b0),
        (col2, val2, rmap2, idx2, dw1, pw1, b1),
        (col1, val1, rmap1, idx1, dw2, pw2, b2),
        (col0, val0, rmap0, idx0, dw3, pw3, b3),
    ]
    for col, val, rmap, idx, dw, pw, b in levels:
        N, S = idx.shape
        Np = _ceil32(N)
        O = pw.shape[0]
        C = dw.shape[0] // S
        pcol = jnp.pad(jnp.take(col, rmap), ((0, Np - N), (0, 0)))
        pval = jnp.pad(jnp.take(val, rmap), ((0, Np - N), (0, 0)))
        pidx = pcol.reshape(-1).astype(jnp.int32)
        wexp = jnp.broadcast_to(pval.reshape(-1)[:, None].astype(f32),
                                (Np * 3, C))
        X = _sc_gather_sum(z, pidx, wexp, None, K=3, D=C, Np=Np, B=B,
                           rowstride=vtab, relu=False)
        Y = _tc_matmul(X, _wcat(pw, dw, S, C, O))
        Yr = Y.reshape(B * Np * S, O)
        bidx = idx.astype(jnp.int32) * S + jnp.arange(S, dtype=jnp.int32)[None]
        bidx = jnp.pad(bidx, ((0, Np - N), (0, 0))).reshape(-1)
        z = _sc_gather_sum(Yr, bidx, None, b.astype(f32), K=S, D=O, Np=Np,
                           B=B, rowstride=Np * S, relu=True)
        vtab = Np

    N, S = idx0.shape
    Opad = 16
    Yh = _tc_matmul(z, _wcat(pwh, dwh, S, 32, 3, Opad=Opad))
    Yhr = Yh.reshape(B * vtab * S, Opad)
    bidxh = (idx0.astype(jnp.int32) * S +
             jnp.arange(S, dtype=jnp.int32)[None]).reshape(-1)
    out = _sc_gather_sum(Yhr, bidxh, None,
                         jnp.pad(bh.astype(f32), (0, Opad - 3)),
                         K=S, D=Opad, Np=N, B=B, rowstride=vtab * S,
                         relu=False)
    return out.reshape(B, N, Opad)[:, :, :3]


def kernel(uv, x, upsample, idx0, idx1, idx2, idx3, col0, col1, col2, col3,
           val0, val1, val2, val3, rmap0, rmap1, rmap2, rmap3,
           dw0, pw0, b0, dw1, pw1, b1, dw2, pw2, b2, dw3, pw3, b3,
           dwh, pwh, bh):
    B = uv.shape[0]
    H = B // 2
    rest = (idx0, idx1, idx2, idx3, col0, col1, col2, col3,
            val0, val1, val2, val3, rmap0, rmap1, rmap2, rmap3,
            dw0, pw0, b0, dw1, pw1, b1, dw2, pw2, b2, dw3, pw3, b3,
            dwh, pwh, bh)
    o1 = _half(uv[:H], x[:H], upsample, *rest)
    o2 = _half(uv[H:], x[H:], upsample, *rest)
    return jnp.concatenate([o1, o2], axis=0)


# speedup vs baseline: 1.3048x; 1.0020x over previous
import jax
import jax.numpy as jnp
from jax import lax
from jax.experimental import pallas as pl
from jax.experimental.pallas import tpu as pltpu
from jax.experimental.pallas import tpu_sc as plsc

_NC = 2
_NW = 32


def _ceil16(n):
    return (n + 15) // 16 * 16


def _sc_gather_sum(table, pidx, wexp, bias, *, K, D, Np, B, rowstride, relu):
    T = 16
    KT = K * T
    nchunks = Np // T
    BG = 2 if nchunks >= 32 else (4 if nchunks >= 16 else 8)
    bsz = B // BG
    P = bsz // 2
    njobs = nchunks * BG
    per = -(-njobs // _NW)
    parts = []
    off = 0
    while off < KT:
        r = min(128, KT - off)
        parts.append((off, r))
        off += r
    weighted = wexp is not None
    has_bias = bias is not None

    scratch = [pltpu.VMEM((KT,), jnp.int32)]
    for _ in range(2):
        for (_o, r) in parts:
            scratch.append(pltpu.VMEM((r,), jnp.int32))
    scratch += [pltpu.VMEM((KT, D), jnp.float32),
                pltpu.VMEM((KT, D), jnp.float32)]
    if weighted:
        scratch.append(pltpu.VMEM((KT, D), jnp.float32))
    if has_bias:
        scratch.append(pltpu.VMEM((D,), jnp.float32))
    scratch += [pltpu.VMEM((T, D), jnp.float32),
                pltpu.VMEM((T, D), jnp.float32)]
    scratch += [pltpu.SemaphoreType.DMA] * 4

    mesh = plsc.VectorSubcoreMesh(core_axis_name="c", subcore_axis_name="s")

    def body(*refs):
        it = iter(refs)
        table_r = next(it)
        pidx_r = next(it)
        wexp_r = next(it) if weighted else None
        bias_r = next(it) if has_bias else None
        out_r = next(it)
        pidxv = next(it)
        idxb = [[next(it) for _ in parts], [next(it) for _ in parts]]
        gbuf = [next(it), next(it)]
        wbuf = next(it) if weighted else None
        biasv = next(it) if has_bias else None
        obuf = [next(it), next(it)]
        sem_g = [next(it), next(it)]
        sem_s = [next(it), next(it)]

        wid = lax.axis_index("s") * _NC + lax.axis_index("c")
        if has_bias:
            pltpu.sync_copy(bias_r, biasv)

        def build_idx(b, u):
            boff = b * rowstride
            for v in range(KT // 16):
                pi = v // 8
                o_local = 16 * v - parts[pi][0]
                idxb[u][pi][pl.ds(o_local, 16)] = \
                    pidxv[pl.ds(16 * v, 16)] + boff

        def fire_gather(u):
            for (o, r), iref in zip(parts, idxb[u]):
                pltpu.async_copy(table_r.at[iref],
                                 gbuf[u].at[pl.ds(o, r)], sem_g[u])

        def drain_gather(u):
            for (o, r), iref in zip(parts, idxb[u]):
                pltpu.make_async_copy(table_r.at[iref],
                                      gbuf[u].at[pl.ds(o, r)],
                                      sem_g[u]).wait()

        def fire_scatter(u, b, n0):
            pltpu.async_copy(obuf[u], out_r.at[pl.ds(b * Np + n0, T)],
                             sem_s[u])

        def drain_scatter(u):
            pltpu.make_async_copy(obuf[u], out_r.at[pl.ds(0, T)],
                                  sem_s[u]).wait()

        def compute(u):
            gb, ob = gbuf[u], obuf[u]

            @plsc.parallel_loop(0, T, 1, unroll=2)
            def _tbody(t):
                row = t * K
                for dd in range(D // 16):
                    o = dd * 16
                    vals = []
                    for k in range(K):
                        g = gb[row + k, pl.ds(o, 16)]
                        if weighted:
                            g = g * wbuf[row + k, pl.ds(o, 16)]
                        vals.append(g)
                    while len(vals) > 1:
                        nxt = [vals[i] + vals[i + 1]
                               for i in range(0, len(vals) - 1, 2)]
                        if len(vals) % 2:
                            nxt.append(vals[-1])
                        vals = nxt
                    acc = vals[0]
                    if has_bias:
                        acc = acc + biasv[pl.ds(o, 16)]
                    if relu:
                        acc = jnp.maximum(acc, 0.0)
                    ob[t, pl.ds(o, 16)] = acc

        def job(jw, _):
            j = jw * _NW + wid

            @pl.when(j < njobs)
            def _():
                ci = j // BG
                bg = j % BG
                n0 = ci * T
                base = bg * bsz
                pltpu.sync_copy(pidx_r.at[pl.ds(n0 * K, KT)], pidxv)
                if weighted:
                    pltpu.sync_copy(wexp_r.at[pl.ds(n0 * K, KT)], wbuf)
                build_idx(base, 0)
                fire_gather(0)

                def pair(bp, _):
                    b0 = base + 2 * bp
                    build_idx(b0 + 1, 1)
                    fire_gather(1)
                    drain_gather(0)

                    @pl.when(bp > 0)
                    def _():
                        drain_scatter(0)

                    compute(0)
                    fire_scatter(0, b0, n0)

                    @pl.when(bp < P - 1)
                    def _():
                        build_idx(b0 + 2, 0)
                        fire_gather(0)

                    drain_gather(1)

                    @pl.when(bp > 0)
                    def _():
                        drain_scatter(1)

                    compute(1)
                    fire_scatter(1, b0 + 1, n0)
                    return 0

                lax.fori_loop(0, P, pair, 0)
                drain_scatter(0)
                drain_scatter(1)

            return 0

        lax.fori_loop(0, per, job, 0)

    args = [table, pidx]
    if weighted:
        args.append(wexp)
    if has_bias:
        args.append(bias)
    fn = pl.kernel(body,
                   out_type=jax.ShapeDtypeStruct((B * Np, D), jnp.float32),
                   mesh=mesh,
                   scratch_types=scratch,
                   compiler_params=pltpu.CompilerParams(
                       use_tc_tiling_on_sc=False))
    return fn(*args)


def _mm_body(x_ref, w_ref, o_ref):
    o_ref[...] = jnp.dot(x_ref[...], w_ref[...],
                         preferred_element_type=jnp.float32)


def _tc_matmul(x, w):
    M, C = x.shape
    KO = w.shape[1]
    BM = 256
    return pl.pallas_call(
        _mm_body,
        grid=(M // BM,),
        in_specs=[pl.BlockSpec((BM, C), lambda i: (i, 0)),
                  pl.BlockSpec((C, KO), lambda i: (0, 0))],
        out_specs=pl.BlockSpec((BM, KO), lambda i: (i, 0)),
        out_shape=jax.ShapeDtypeStruct((M, KO), jnp.float32),
    )(x, w)


def _bilin_body(uv_ref, x_ref, up_ref, o_ref):
    uvb = uv_ref[0]
    im = x_ref[0]
    up = up_ref[...]
    uvc = jnp.clip((uvb - 0.5) * 2.0, -1.0, 1.0)
    gx = (uvc[:, 0:1] + 1.0) * 1.5
    gy = (uvc[:, 1:2] + 1.0) * 1.5
    x0 = jnp.floor(gx)
    y0 = jnp.floor(gy)
    wa = (x0 + 1.0 - gx) * (y0 + 1.0 - gy)
    wb = (x0 + 1.0 - gx) * (gy - y0)
    wc = (gx - x0) * (y0 + 1.0 - gy)
    wd = (gx - x0) * (gy - y0)
    x0i = x0.astype(jnp.int32)
    y0i = y0.astype(jnp.int32)
    cell = lax.broadcasted_iota(jnp.int32, (21, 16), 1)

    def oh(xi, yi, w):
        c = yi * 4 + xi
        valid = (xi >= 0) & (xi <= 3) & (yi >= 0) & (yi <= 3)
        return jnp.where((cell == c) & valid, w, 0.0)

    wgt = (oh(x0i, y0i, wa) + oh(x0i, y0i + 1, wb) +
           oh(x0i + 1, y0i, wc) + oh(x0i + 1, y0i + 1, wd))
    t1 = lax.dot_general(wgt, im, (((1,), (1,)), ((), ())),
                         preferred_element_type=jnp.float32)
    o_ref[0] = jnp.dot(up, t1, preferred_element_type=jnp.float32)


def _tc_bilinear_upsample(uv, x16, upsample):
    B = uv.shape[0]
    return pl.pallas_call(
        _bilin_body,
        grid=(B,),
        in_specs=[pl.BlockSpec((1, 21, 2), lambda b: (b, 0, 0)),
                  pl.BlockSpec((1, 256, 16), lambda b: (b, 0, 0)),
                  pl.BlockSpec((98, 21), lambda b: (0, 0))],
        out_specs=pl.BlockSpec((1, 98, 256), lambda b: (b, 0, 0)),
        out_shape=jax.ShapeDtypeStruct((B, 98, 256), jnp.float32),
    )(uv, x16, upsample)


def _wcat(pw, dw, S, C, O, Opad=None):
    pwdw = (pw * dw[None, :]).reshape(O, S, C)
    w = jnp.transpose(pwdw, (2, 1, 0))
    if Opad is not None and Opad != O:
        w = jnp.pad(w, ((0, 0), (0, 0), (0, Opad - O)))
        O = Opad
    return w.reshape(C, S * O)


def _half(uv, x, upsample, idx0, idx1, idx2, idx3, col0, col1, col2, col3,
          val0, val1, val2, val3, rmap0, rmap1, rmap2, rmap3,
          dw0, pw0, b0, dw1, pw1, b1, dw2, pw2, b2, dw3, pw3, b3,
          dwh, pwh, bh):
    B = uv.shape[0]
    f32 = jnp.float32

    z = _tc_bilinear_upsample(uv, x.reshape(B, 256, 16), upsample)
    z = z.reshape(B * 98, 256)
    vtab = 98

    levels = [
        (col3, val3, rmap3, idx3, dw0, pw0, b0),
        (col2, val2, rmap2, idx2, dw1, pw1, b1),
        (col1, val1, rmap1, idx1, dw2, pw2, b2),
        (col0, val0, rmap0, idx0, dw3, pw3, b3),
    ]
    for col, val, rmap, idx, dw, pw, b in levels:
        N, S = idx.shape
        Np = _ceil16(N)
        O = pw.shape[0]
        C = dw.shape[0] // S
        pcol = jnp.pad(jnp.take(col, rmap), ((0, Np - N), (0, 0)))
        pval = jnp.pad(jnp.take(val, rmap), ((0, Np - N), (0, 0)))
        pidx = pcol.reshape(-1).astype(jnp.int32)
        wexp = jnp.broadcast_to(pval.reshape(-1)[:, None].astype(f32),
                                (Np * 3, C))
        X = _sc_gather_sum(z, pidx, wexp, None, K=3, D=C, Np=Np, B=B,
                           rowstride=vtab, relu=False)
        Y = _tc_matmul(X, _wcat(pw, dw, S, C, O))
        Yr = Y.reshape(B * Np * S, O)
        bidx = idx.astype(jnp.int32) * S + jnp.arange(S, dtype=jnp.int32)[None]
        bidx = jnp.pad(bidx, ((0, Np - N), (0, 0))).reshape(-1)
        z = _sc_gather_sum(Yr, bidx, None, b.astype(f32), K=S, D=O, Np=Np,
                           B=B, rowstride=Np * S, relu=True)
        vtab = Np

    N, S = idx0.shape
    Opad = 16
    Yh = _tc_matmul(z, _wcat(pwh, dwh, S, 32, 3, Opad=Opad))
    Yhr = Yh.reshape(B * vtab * S, Opad)
    bidxh = (idx0.astype(jnp.int32) * S +
             jnp.arange(S, dtype=jnp.int32)[None]).reshape(-1)
    out = _sc_gather_sum(Yhr, bidxh, None,
                         jnp.pad(bh.astype(f32), (0, Opad - 3)),
                         K=S, D=Opad, Np=N, B=B, rowstride=vtab * S,
                         relu=False)
    return out.reshape(B, N, Opad)[:, :, :3]


def kernel(uv, x, upsample, idx0, idx1, idx2, idx3, col0, col1, col2, col3,
           val0, val1, val2, val3, rmap0, rmap1, rmap2, rmap3,
           dw0, pw0, b0, dw1, pw1, b1, dw2, pw2, b2, dw3, pw3, b3,
           dwh, pwh, bh):
    B = uv.shape[0]
    H = B // 2
    rest = (idx0, idx1, idx2, idx3, col0, col1, col2, col3,
            val0, val1, val2, val3, rmap0, rmap1, rmap2, rmap3,
            dw0, pw0, b0, dw1, pw1, b1, dw2, pw2, b2, dw3, pw3, b3,
            dwh, pwh, bh)
    o1 = _half(uv[:H], x[:H], upsample, *rest)
    o2 = _half(uv[H:], x[H:], upsample, *rest)
    return jnp.concatenate([o1, o2], axis=0)


# speedup vs baseline: 1.3207x; 1.0122x over previous
import jax
import jax.numpy as jnp
from jax import lax
from jax.experimental import pallas as pl
from jax.experimental.pallas import tpu as pltpu
from jax.experimental.pallas import tpu_sc as plsc

_NC = 2
_NW = 32


def _ceil16(n):
    return (n + 15) // 16 * 16


def _sc_gather_sum(table, pidx, wexp, bias, *, K, D, Np, B, rowstride, relu):
    T = 16
    KT = K * T
    nchunks = Np // T
    BG = 2 if nchunks >= 32 else (4 if nchunks >= 16 else 8)
    bsz = B // BG
    P = bsz // 2
    njobs = nchunks * BG
    per = -(-njobs // _NW)
    parts = []
    off = 0
    while off < KT:
        r = min(128, KT - off)
        parts.append((off, r))
        off += r
    weighted = wexp is not None
    has_bias = bias is not None

    scratch = [pltpu.VMEM((KT,), jnp.int32)]
    for _ in range(2):
        for (_o, r) in parts:
            scratch.append(pltpu.VMEM((r,), jnp.int32))
    scratch += [pltpu.VMEM((KT, D), jnp.float32),
                pltpu.VMEM((KT, D), jnp.float32)]
    if weighted:
        scratch.append(pltpu.VMEM((KT, 16), jnp.float32))
    if has_bias:
        scratch.append(pltpu.VMEM((D,), jnp.float32))
    scratch += [pltpu.VMEM((T, D), jnp.float32),
                pltpu.VMEM((T, D), jnp.float32)]
    scratch += [pltpu.SemaphoreType.DMA] * 4

    mesh = plsc.VectorSubcoreMesh(core_axis_name="c", subcore_axis_name="s")

    def body(*refs):
        it = iter(refs)
        table_r = next(it)
        pidx_r = next(it)
        wexp_r = next(it) if weighted else None
        bias_r = next(it) if has_bias else None
        out_r = next(it)
        pidxv = next(it)
        idxb = [[next(it) for _ in parts], [next(it) for _ in parts]]
        gbuf = [next(it), next(it)]
        wbuf = next(it) if weighted else None
        biasv = next(it) if has_bias else None
        obuf = [next(it), next(it)]
        sem_g = [next(it), next(it)]
        sem_s = [next(it), next(it)]

        wid = lax.axis_index("s") * _NC + lax.axis_index("c")
        if has_bias:
            pltpu.sync_copy(bias_r, biasv)

        def build_idx(b, u):
            boff = b * rowstride
            for v in range(KT // 16):
                pi = v // 8
                o_local = 16 * v - parts[pi][0]
                idxb[u][pi][pl.ds(o_local, 16)] = \
                    pidxv[pl.ds(16 * v, 16)] + boff

        def fire_gather(u):
            for (o, r), iref in zip(parts, idxb[u]):
                pltpu.async_copy(table_r.at[iref],
                                 gbuf[u].at[pl.ds(o, r)], sem_g[u])

        def drain_gather(u):
            for (o, r), iref in zip(parts, idxb[u]):
                pltpu.make_async_copy(table_r.at[iref],
                                      gbuf[u].at[pl.ds(o, r)],
                                      sem_g[u]).wait()

        def fire_scatter(u, b, n0):
            pltpu.async_copy(obuf[u], out_r.at[pl.ds(b * Np + n0, T)],
                             sem_s[u])

        def drain_scatter(u):
            pltpu.make_async_copy(obuf[u], out_r.at[pl.ds(0, T)],
                                  sem_s[u]).wait()

        def compute(u):
            gb, ob = gbuf[u], obuf[u]
            if has_bias:
                bvecs = [biasv[pl.ds(dd * 16, 16)] for dd in range(D // 16)]

            @plsc.parallel_loop(0, T, 1, unroll=2)
            def _tbody(t):
                row = t * K
                if weighted:
                    ws = [wbuf[row + k, pl.ds(0, 16)] for k in range(K)]
                for dd in range(D // 16):
                    o = dd * 16
                    vals = []
                    for k in range(K):
                        g = gb[row + k, pl.ds(o, 16)]
                        if weighted:
                            g = g * ws[k]
                        vals.append(g)
                    while len(vals) > 1:
                        nxt = [vals[i] + vals[i + 1]
                               for i in range(0, len(vals) - 1, 2)]
                        if len(vals) % 2:
                            nxt.append(vals[-1])
                        vals = nxt
                    acc = vals[0]
                    if has_bias:
                        acc = acc + bvecs[dd]
                    if relu:
                        acc = jnp.maximum(acc, 0.0)
                    ob[t, pl.ds(o, 16)] = acc

        def job(jw, _):
            j = jw * _NW + wid

            @pl.when(j < njobs)
            def _():
                ci = j // BG
                bg = j % BG
                n0 = ci * T
                base = bg * bsz
                pltpu.sync_copy(pidx_r.at[pl.ds(n0 * K, KT)], pidxv)
                if weighted:
                    pltpu.sync_copy(wexp_r.at[pl.ds(n0 * K, KT)], wbuf)
                build_idx(base, 0)
                fire_gather(0)

                def pair(bp, _):
                    b0 = base + 2 * bp
                    build_idx(b0 + 1, 1)
                    fire_gather(1)
                    drain_gather(0)

                    @pl.when(bp > 0)
                    def _():
                        drain_scatter(0)

                    compute(0)
                    fire_scatter(0, b0, n0)

                    @pl.when(bp < P - 1)
                    def _():
                        build_idx(b0 + 2, 0)
                        fire_gather(0)

                    drain_gather(1)

                    @pl.when(bp > 0)
                    def _():
                        drain_scatter(1)

                    compute(1)
                    fire_scatter(1, b0 + 1, n0)
                    return 0

                lax.fori_loop(0, P, pair, 0)
                drain_scatter(0)
                drain_scatter(1)

            return 0

        lax.fori_loop(0, per, job, 0)

    args = [table, pidx]
    if weighted:
        args.append(wexp)
    if has_bias:
        args.append(bias)
    fn = pl.kernel(body,
                   out_type=jax.ShapeDtypeStruct((B * Np, D), jnp.float32),
                   mesh=mesh,
                   scratch_types=scratch,
                   compiler_params=pltpu.CompilerParams(
                       use_tc_tiling_on_sc=False))
    return fn(*args)


def _mm_body(x_ref, w_ref, o_ref):
    o_ref[...] = jnp.dot(x_ref[...], w_ref[...],
                         preferred_element_type=jnp.float32)


def _tc_matmul(x, w):
    M, C = x.shape
    KO = w.shape[1]
    BM = 256
    return pl.pallas_call(
        _mm_body,
        grid=(M // BM,),
        in_specs=[pl.BlockSpec((BM, C), lambda i: (i, 0)),
                  pl.BlockSpec((C, KO), lambda i: (0, 0))],
        out_specs=pl.BlockSpec((BM, KO), lambda i: (i, 0)),
        out_shape=jax.ShapeDtypeStruct((M, KO), jnp.float32),
    )(x, w)


def _bilin_body(uv_ref, x_ref, up_ref, o_ref):
    uvb = uv_ref[0]
    im = x_ref[0]
    up = up_ref[...]
    uvc = jnp.clip((uvb - 0.5) * 2.0, -1.0, 1.0)
    gx = (uvc[:, 0:1] + 1.0) * 1.5
    gy = (uvc[:, 1:2] + 1.0) * 1.5
    x0 = jnp.floor(gx)
    y0 = jnp.floor(gy)
    wa = (x0 + 1.0 - gx) * (y0 + 1.0 - gy)
    wb = (x0 + 1.0 - gx) * (gy - y0)
    wc = (gx - x0) * (y0 + 1.0 - gy)
    wd = (gx - x0) * (gy - y0)
    x0i = x0.astype(jnp.int32)
    y0i = y0.astype(jnp.int32)
    cell = lax.broadcasted_iota(jnp.int32, (21, 16), 1)

    def oh(xi, yi, w):
        c = yi * 4 + xi
        valid = (xi >= 0) & (xi <= 3) & (yi >= 0) & (yi <= 3)
        return jnp.where((cell == c) & valid, w, 0.0)

    wgt = (oh(x0i, y0i, wa) + oh(x0i, y0i + 1, wb) +
           oh(x0i + 1, y0i, wc) + oh(x0i + 1, y0i + 1, wd))
    t1 = lax.dot_general(wgt, im, (((1,), (1,)), ((), ())),
                         preferred_element_type=jnp.float32)
    o_ref[0] = jnp.dot(up, t1, preferred_element_type=jnp.float32)


def _tc_bilinear_upsample(uv, x16, upsample):
    B = uv.shape[0]
    return pl.pallas_call(
        _bilin_body,
        grid=(B,),
        in_specs=[pl.BlockSpec((1, 21, 2), lambda b: (b, 0, 0)),
                  pl.BlockSpec((1, 256, 16), lambda b: (b, 0, 0)),
                  pl.BlockSpec((98, 21), lambda b: (0, 0))],
        out_specs=pl.BlockSpec((1, 98, 256), lambda b: (b, 0, 0)),
        out_shape=jax.ShapeDtypeStruct((B, 98, 256), jnp.float32),
    )(uv, x16, upsample)


def _wcat(pw, dw, S, C, O, Opad=None):
    pwdw = (pw * dw[None, :]).reshape(O, S, C)
    w = jnp.transpose(pwdw, (2, 1, 0))
    if Opad is not None and Opad != O:
        w = jnp.pad(w, ((0, 0), (0, 0), (0, Opad - O)))
        O = Opad
    return w.reshape(C, S * O)


def _half(uv, x, upsample, idx0, idx1, idx2, idx3, col0, col1, col2, col3,
          val0, val1, val2, val3, rmap0, rmap1, rmap2, rmap3,
          dw0, pw0, b0, dw1, pw1, b1, dw2, pw2, b2, dw3, pw3, b3,
          dwh, pwh, bh):
    B = uv.shape[0]
    f32 = jnp.float32

    z = _tc_bilinear_upsample(uv, x.reshape(B, 256, 16), upsample)
    z = z.reshape(B * 98, 256)
    vtab = 98

    levels = [
        (col3, val3, rmap3, idx3, dw0, pw0, b0),
        (col2, val2, rmap2, idx2, dw1, pw1, b1),
        (col1, val1, rmap1, idx1, dw2, pw2, b2),
        (col0, val0, rmap0, idx0, dw3, pw3, b3),
    ]
    for col, val, rmap, idx, dw, pw, b in levels:
        N, S = idx.shape
        Np = _ceil16(N)
        O = pw.shape[0]
        C = dw.shape[0] // S
        pcol = jnp.pad(jnp.take(col, rmap), ((0, Np - N), (0, 0)))
        pval = jnp.pad(jnp.take(val, rmap), ((0, Np - N), (0, 0)))
        pidx = pcol.reshape(-1).astype(jnp.int32)
        wexp = jnp.broadcast_to(pval.reshape(-1)[:, None].astype(f32),
                                (Np * 3, 16))
        X = _sc_gather_sum(z, pidx, wexp, None, K=3, D=C, Np=Np, B=B,
                           rowstride=vtab, relu=False)
        Y = _tc_matmul(X, _wcat(pw, dw, S, C, O))
        Yr = Y.reshape(B * Np * S, O)
        bidx = idx.astype(jnp.int32) * S + jnp.arange(S, dtype=jnp.int32)[None]
        bidx = jnp.pad(bidx, ((0, Np - N), (0, 0))).reshape(-1)
        z = _sc_gather_sum(Yr, bidx, None, b.astype(f32), K=S, D=O, Np=Np,
                           B=B, rowstride=Np * S, relu=True)
        vtab = Np

    N, S = idx0.shape
    Opad = 16
    Yh = _tc_matmul(z, _wcat(pwh, dwh, S, 32, 3, Opad=Opad))
    Yhr = Yh.reshape(B * vtab * S, Opad)
    bidxh = (idx0.astype(jnp.int32) * S +
             jnp.arange(S, dtype=jnp.int32)[None]).reshape(-1)
    out = _sc_gather_sum(Yhr, bidxh, None,
                         jnp.pad(bh.astype(f32), (0, Opad - 3)),
                         K=S, D=Opad, Np=N, B=B, rowstride=vtab * S,
                         relu=False)
    return out.reshape(B, N, Opad)[:, :, :3]


def kernel(uv, x, upsample, idx0, idx1, idx2, idx3, col0, col1, col2, col3,
           val0, val1, val2, val3, rmap0, rmap1, rmap2, rmap3,
           dw0, pw0, b0, dw1, pw1, b1, dw2, pw2, b2, dw3, pw3, b3,
           dwh, pwh, bh):
    B = uv.shape[0]
    H = B // 2
    rest = (idx0, idx1, idx2, idx3, col0, col1, col2, col3,
            val0, val1, val2, val3, rmap0, rmap1, rmap2, rmap3,
            dw0, pw0, b0, dw1, pw1, b1, dw2, pw2, b2, dw3, pw3, b3,
            dwh, pwh, bh)
    o1 = _half(uv[:H], x[:H], upsample, *rest)
    o2 = _half(uv[H:], x[H:], upsample, *rest)
    return jnp.concatenate([o1, o2], axis=0)
